# baseline (speedup 1.0000x reference)
import functools

import jax
import jax.numpy as jnp
from jax import lax
from jax.experimental import pallas as pl
from jax.experimental.pallas import tpu as pltpu
from jax.experimental.pallas import tpu_sc as plsc

EMB_DIM = 128


def _make_sc_lookup(num_rows, emb_dim):
    mesh = plsc.ScalarSubcoreMesh(axis_name="c", num_cores=1)

    @functools.partial(
        pl.kernel,
        mesh=mesh,
        out_type=jax.ShapeDtypeStruct((1, emb_dim), jnp.float32),
        scratch_types=[
            pltpu.SMEM((1,), jnp.int32),
        ],
    )
    def _lookup(table_hbm, idx_hbm, out_hbm, idx_s):
        pltpu.sync_copy(idx_hbm, idx_s)
        s = idx_s[0]
        pltpu.sync_copy(table_hbm.at[pl.ds(s, 1)], out_hbm)

    return _lookup


def kernel(genre_emb, genre_idx):
    idx = jnp.atleast_1d(jnp.asarray(genre_idx, jnp.int32))
    lookup = _make_sc_lookup(genre_emb.shape[0], genre_emb.shape[1])
    return lookup(genre_emb, idx)
